# baseline (speedup 1.0000x reference)
import functools

import jax
import jax.numpy as jnp
from jax import lax
from jax.experimental import pallas as pl
from jax.experimental.pallas import tpu as pltpu
from jax.experimental.pallas import tpu_sc as plsc

_B = 16384
_C = 1000
_S = 12800

_NW = 32
_RPW = (_B - _S) // _NW
_RB = 16
_NCH = _RPW // _RB

_BR = 1280
_NBUF = 4
_NCHUNK = _S // _BR


def _sc_body(p_hbm, y_hbm, out_hbm, bufA, bufB, yv, accv, sd0, sd1):
    w = lax.axis_index("s") * 2 + lax.axis_index("c")
    row0 = _S + w * _RPW
    sems = (sd0, sd1)

    bufs = (bufA, bufB)

    def copy(t, b):
        return pltpu.make_async_copy(
            p_hbm.at[pl.ds(row0 + t * _RB, _RB), :], bufs[b], sems[b]
        )

    copy(0, 0).start()
    pltpu.sync_copy(y_hbm.at[pl.ds(row0, _RPW)], yv)

    lane = lax.broadcasted_iota(jnp.int32, (16,), 0)
    zeros16 = jnp.zeros((16,), jnp.float32)

    a0 = zeros16
    a1 = zeros16
    gacc = zeros16
    for t in range(_NCH):
        b = t % 2
        if t + 1 < _NCH:
            copy(t + 1, 1 - b).start()
        copy(t, b).wait()

        for r in range(_RB):
            lr = t * _RB + r
            ybc = lax.gather(
                yv[pl.ds((lr // 16) * 16, 16)],
                jnp.full((16, 1), lr % 16, jnp.int32),
                lax.GatherDimensionNumbers(
                    offset_dims=(),
                    collapsed_slice_dims=(0,),
                    start_index_map=(0,),
                ),
                slice_sizes=(1,),
                mode=lax.GatherScatterMode.PROMISE_IN_BOUNDS,
            )

            def ibody(e, c, _b=b, _r=r, _ybc=ybc):
                a, g = c
                v = bufs[_b][_r, pl.ds(e, 16)]
                hit = (lane + e) == _ybc
                return a + v * v, g + jnp.where(hit, v, zeros16)

            a0, gacc = plsc.parallel_loop(
                0, 992, step=16, unroll=16, carry=(a0, gacc)
            )(ibody)
            vt = bufs[b][r, pl.ds(984, 16)]
            vt = jnp.where(lane >= 8, vt, zeros16)
            a0 = a0 + vt * vt
            gacc = gacc + jnp.where((lane + 984) == ybc, vt, zeros16)

    accv[pl.ds(0, 16)] = a0 + a1
    accv[pl.ds(16, 16)] = gacc
    pltpu.sync_copy(accv, out_hbm.at[w])


def _sc_partials(probs, y32):
    mesh = plsc.VectorSubcoreMesh(core_axis_name="c", subcore_axis_name="s")
    run = functools.partial(
        pl.kernel,
        mesh=mesh,
        out_type=jax.ShapeDtypeStruct((_NW, 32), jnp.float32),
        scratch_types=[
            pltpu.VMEM((_RB, _C), jnp.float32),
            pltpu.VMEM((_RB, _C), jnp.float32),
            pltpu.VMEM((_RPW,), jnp.int32),
            pltpu.VMEM((32,), jnp.float32),
            pltpu.SemaphoreType.DMA,
            pltpu.SemaphoreType.DMA,
        ],
    )(_sc_body)
    return run(probs, y32)


def _tc_body(y_ref, p_hbm, out_ref, b0, b1, b2, b3, s0, s1, s2, s3):
    bufs = (b0, b1, b2, b3)
    sems = (s0, s1, s2, s3)

    def copy(i, slot):
        return pltpu.make_async_copy(
            p_hbm.at[pl.ds(i * _BR, _BR), :], bufs[slot], sems[slot]
        )

    for s in range(_NBUF):
        copy(s, s).start()

    col = jax.lax.broadcasted_iota(jnp.int32, (_BR, _C), 1)
    acc = jnp.float32(0.0)
    for i in range(_NCHUNK):
        slot = i % _NBUF
        copy(i, slot).wait()
        p = bufs[slot][...]
        yb = y_ref[pl.ds(i * _BR, _BR), :]
        acc += jnp.sum(p * p) - 2.0 * jnp.sum(jnp.where(col == yb, p, 0.0))
        if i + _NBUF < _NCHUNK:
            copy(i + _NBUF, slot).start()

    out_ref[0, 0] = acc


def _combine_body(tc_ref, sc_ref, out_ref):
    sc = sc_ref[...]
    total = tc_ref[0, 0] + jnp.sum(sc[:, 0:16]) - 2.0 * jnp.sum(sc[:, 16:32])
    out_ref[0, 0] = (total + jnp.float32(_B)) / jnp.float32(_B)


def kernel(probs, y):
    y32 = y.astype(jnp.int32)
    tc_part = pl.pallas_call(
        _tc_body,
        in_specs=[
            pl.BlockSpec(memory_space=pltpu.VMEM),
            pl.BlockSpec(memory_space=pl.ANY),
        ],
        out_specs=pl.BlockSpec(memory_space=pltpu.SMEM),
        out_shape=jax.ShapeDtypeStruct((1, 1), jnp.float32),
        scratch_shapes=(
            [pltpu.VMEM((_BR, _C), jnp.float32) for _ in range(_NBUF)]
            + [pltpu.SemaphoreType.DMA for _ in range(_NBUF)]
        ),
    )(y32[:_S].reshape(_S, 1), probs)
    partials = _sc_partials(probs, y32)
    out = pl.pallas_call(
        _combine_body,
        in_specs=[
            pl.BlockSpec(memory_space=pltpu.SMEM),
            pl.BlockSpec(memory_space=pltpu.VMEM),
        ],
        out_specs=pl.BlockSpec(memory_space=pltpu.SMEM),
        out_shape=jax.ShapeDtypeStruct((1, 1), jnp.float32),
    )(tc_part, partials)
    return out[0, 0]


# speedup vs baseline: 1.0259x; 1.0259x over previous
import functools

import jax
import jax.numpy as jnp
from jax import lax
from jax.experimental import pallas as pl
from jax.experimental.pallas import tpu as pltpu
from jax.experimental.pallas import tpu_sc as plsc

_B = 16384
_C = 1000
_S = 14336

_NW = 32
_RPW = (_B - _S) // _NW
_RB = 16
_NCH = _RPW // _RB

_BR = 1024
_NBUF = 4
_NCHUNK = _S // _BR


def _sc_body(p_hbm, y_hbm, out_hbm, bufA, bufB, yv, accv, sd0, sd1):
    w = lax.axis_index("s") * 2 + lax.axis_index("c")
    row0 = _S + w * _RPW
    sems = (sd0, sd1)

    bufs = (bufA, bufB)

    def copy(t, b):
        return pltpu.make_async_copy(
            p_hbm.at[pl.ds(row0 + t * _RB, _RB), :], bufs[b], sems[b]
        )

    copy(0, 0).start()
    pltpu.sync_copy(y_hbm.at[pl.ds(row0, _RPW)], yv)

    lane = lax.broadcasted_iota(jnp.int32, (16,), 0)
    zeros16 = jnp.zeros((16,), jnp.float32)

    a0 = zeros16
    a1 = zeros16
    gacc = zeros16
    for t in range(_NCH):
        b = t % 2
        if t + 1 < _NCH:
            copy(t + 1, 1 - b).start()
        copy(t, b).wait()

        for r in range(_RB):
            lr = t * _RB + r
            ybc = lax.gather(
                yv[pl.ds((lr // 16) * 16, 16)],
                jnp.full((16, 1), lr % 16, jnp.int32),
                lax.GatherDimensionNumbers(
                    offset_dims=(),
                    collapsed_slice_dims=(0,),
                    start_index_map=(0,),
                ),
                slice_sizes=(1,),
                mode=lax.GatherScatterMode.PROMISE_IN_BOUNDS,
            )

            def ibody(e, c, _b=b, _r=r, _ybc=ybc):
                a, g = c
                v = bufs[_b][_r, pl.ds(e, 16)]
                hit = (lane + e) == _ybc
                return a + v * v, g + jnp.where(hit, v, zeros16)

            a0, gacc = plsc.parallel_loop(
                0, 992, step=16, unroll=16, carry=(a0, gacc)
            )(ibody)
            vt = bufs[b][r, pl.ds(984, 16)]
            vt = jnp.where(lane >= 8, vt, zeros16)
            a0 = a0 + vt * vt
            gacc = gacc + jnp.where((lane + 984) == ybc, vt, zeros16)

    accv[pl.ds(0, 16)] = a0 + a1
    accv[pl.ds(16, 16)] = gacc
    pltpu.sync_copy(accv, out_hbm.at[w])


def _sc_partials(probs, y32):
    mesh = plsc.VectorSubcoreMesh(core_axis_name="c", subcore_axis_name="s")
    run = functools.partial(
        pl.kernel,
        mesh=mesh,
        out_type=jax.ShapeDtypeStruct((_NW, 32), jnp.float32),
        scratch_types=[
            pltpu.VMEM((_RB, _C), jnp.float32),
            pltpu.VMEM((_RB, _C), jnp.float32),
            pltpu.VMEM((_RPW,), jnp.int32),
            pltpu.VMEM((32,), jnp.float32),
            pltpu.SemaphoreType.DMA,
            pltpu.SemaphoreType.DMA,
        ],
    )(_sc_body)
    return run(probs, y32)


def _tc_body(y_ref, p_hbm, out_ref, b0, b1, b2, b3, s0, s1, s2, s3):
    bufs = (b0, b1, b2, b3)
    sems = (s0, s1, s2, s3)

    def copy(i, slot):
        return pltpu.make_async_copy(
            p_hbm.at[pl.ds(i * _BR, _BR), :], bufs[slot], sems[slot]
        )

    for s in range(_NBUF):
        copy(s, s).start()

    col = jax.lax.broadcasted_iota(jnp.int32, (_BR, _C), 1)
    acc = jnp.float32(0.0)
    for i in range(_NCHUNK):
        slot = i % _NBUF
        copy(i, slot).wait()
        p = bufs[slot][...]
        yb = y_ref[pl.ds(i * _BR, _BR), :]
        acc += jnp.sum(p * p) - 2.0 * jnp.sum(jnp.where(col == yb, p, 0.0))
        if i + _NBUF < _NCHUNK:
            copy(i + _NBUF, slot).start()

    out_ref[0, 0] = acc


def _combine_body(tc_ref, sc_ref, out_ref):
    sc = sc_ref[...]
    total = tc_ref[0, 0] + jnp.sum(sc[:, 0:16]) - 2.0 * jnp.sum(sc[:, 16:32])
    out_ref[0, 0] = (total + jnp.float32(_B)) / jnp.float32(_B)


def kernel(probs, y):
    y32 = y.astype(jnp.int32)
    tc_part = pl.pallas_call(
        _tc_body,
        in_specs=[
            pl.BlockSpec(memory_space=pltpu.VMEM),
            pl.BlockSpec(memory_space=pl.ANY),
        ],
        out_specs=pl.BlockSpec(memory_space=pltpu.SMEM),
        out_shape=jax.ShapeDtypeStruct((1, 1), jnp.float32),
        scratch_shapes=(
            [pltpu.VMEM((_BR, _C), jnp.float32) for _ in range(_NBUF)]
            + [pltpu.SemaphoreType.DMA for _ in range(_NBUF)]
        ),
    )(y32[:_S].reshape(_S, 1), probs)
    partials = _sc_partials(probs, y32)
    out = pl.pallas_call(
        _combine_body,
        in_specs=[
            pl.BlockSpec(memory_space=pltpu.SMEM),
            pl.BlockSpec(memory_space=pltpu.VMEM),
        ],
        out_specs=pl.BlockSpec(memory_space=pltpu.SMEM),
        out_shape=jax.ShapeDtypeStruct((1, 1), jnp.float32),
    )(tc_part, partials)
    return out[0, 0]
